# Initial kernel scaffold; baseline (speedup 1.0000x reference)
import jax
import jax.numpy as jnp
from jax.experimental import pallas as pl


def kernel(x, neuron_emb, W_attn, b_attn, W_know, b_know, W_tau_attn, b_tau_attn, W_tau_know, b_tau_know, cluster_emb_qk, cluster_emb_v, cluster_emb_know):
    raise NotImplementedError("write your pallas kernel here")


# speedup vs baseline: 40.7678x; 40.7678x over previous
import functools

import jax
import jax.numpy as jnp
from jax.experimental import pallas as pl

D_MODEL = 1024
D_SPACE = 64
KC = 8
MAX_K = 32
KEEP = 0.9
CLUSTER_SIZE = 32
TS = 256

_HI = jax.lax.Precision.HIGHEST


def _gate_block_kernel(x_ref, wgt_ref, bgt_ref, emb_ref, ce_ref, ex_ref,
                       out_ref, accc_ref, accn_ref, *, n_clusters, n_neurons):
    step = pl.program_id(0)

    @pl.when(step == 0)
    def _init():
        accc_ref[...] = jnp.zeros_like(accc_ref)
        accn_ref[...] = jnp.zeros_like(accn_ref)

    x = x_ref[...]
    ht = jax.lax.dot_general(x, wgt_ref[...], (((1,), (0,)), ((), ())),
                             preferred_element_type=jnp.float32)
    ht = ht + bgt_ref[...]
    h = ht[:, :D_SPACE] * (1.0 / KEEP)
    tau = ht[:, D_SPACE:D_SPACE + 1]

    ce = ce_ref[...]
    inv_c = 1.0 / (jnp.sqrt(jnp.sum(ce * ce, axis=0, keepdims=True)) + 1e-08)
    cs = jax.lax.dot_general(h, ce * inv_c, (((1,), (0,)), ((), ())),
                             preferred_element_type=jnp.float32)

    cmax = jnp.max(cs, axis=-1, keepdims=True)
    ce_exp = jnp.exp(cs - cmax)
    probs = ce_exp / jnp.sum(ce_exp, axis=-1, keepdims=True)
    accc_ref[...] += jnp.broadcast_to(
        jnp.sum(probs, axis=0, keepdims=True), accc_ref.shape)

    w = cs
    t8 = None
    for _ in range(KC):
        t8 = jnp.max(w, axis=-1, keepdims=True)
        w = jnp.where(w == t8, -1e30, w)
    act = (cs >= t8).astype(jnp.float32)

    act_n = jax.lax.dot_general(act, ex_ref[...], (((1,), (0,)), ((), ())),
                                preferred_element_type=jnp.float32) > 0.5

    emb = emb_ref[...]
    inv_n = 1.0 / (jnp.sqrt(jnp.sum(emb * emb, axis=0, keepdims=True)) + 1e-08)
    scores = jax.lax.dot_general(h, emb * inv_n, (((1,), (0,)), ((), ())),
                                 preferred_element_type=jnp.float32)

    raw = scores - tau
    gate = jnp.where(raw > 0, raw, 1e-08 * jnp.exp(raw))
    eg = jnp.exp(gate) - 1.0
    eg = jnp.where(act_n, eg, 0.0)

    w = eg
    m1 = jnp.max(w, axis=-1, keepdims=True)
    thr = m1
    w = jnp.where(w == thr, -1e30, w)
    for _ in range(MAX_K - 1):
        thr = jnp.max(w, axis=-1, keepdims=True)
        w = jnp.where(w == thr, -1e30, w)

    kept = jnp.where(eg >= thr, eg, 0.0)
    gsum = jnp.sum(kept, axis=-1, keepdims=True) + 1e-08
    out = kept * (jnp.tanh(m1) / gsum)
    out_ref[...] = out
    accn_ref[...] += jnp.broadcast_to(
        jnp.sum(out, axis=0, keepdims=True), accn_ref.shape)


def _run_gate(x2d, wgt, bgt, emb_t, ce_t, ex, n_clusters, n_neurons):
    tokens = x2d.shape[0]
    grid = tokens // TS
    kern = functools.partial(_gate_block_kernel, n_clusters=n_clusters,
                             n_neurons=n_neurons)
    out, accc, accn = pl.pallas_call(
        kern,
        grid=(grid,),
        in_specs=[
            pl.BlockSpec((TS, D_MODEL), lambda i: (i, 0)),
            pl.BlockSpec((D_MODEL, 128), lambda i: (0, 0)),
            pl.BlockSpec((1, 128), lambda i: (0, 0)),
            pl.BlockSpec((D_SPACE, n_neurons), lambda i: (0, 0)),
            pl.BlockSpec((D_SPACE, n_clusters), lambda i: (0, 0)),
            pl.BlockSpec((n_clusters, n_neurons), lambda i: (0, 0)),
        ],
        out_specs=[
            pl.BlockSpec((TS, n_neurons), lambda i: (i, 0)),
            pl.BlockSpec((8, n_clusters), lambda i: (0, 0)),
            pl.BlockSpec((8, n_neurons), lambda i: (0, 0)),
        ],
        out_shape=[
            jax.ShapeDtypeStruct((tokens, n_neurons), jnp.float32),
            jax.ShapeDtypeStruct((8, n_clusters), jnp.float32),
            jax.ShapeDtypeStruct((8, n_neurons), jnp.float32),
        ],
    )(x2d, wgt, bgt, emb_t, ce_t, ex)
    return out, accc[0], accn[0]


def _aux(freq_sum, tokens, n):
    freq = freq_sum / tokens
    return ((freq - 1.0 / n) ** 2).sum() * n


def kernel(x, neuron_emb, W_attn, b_attn, W_know, b_know, W_tau_attn,
           b_tau_attn, W_tau_know, b_tau_know, cluster_emb_qk, cluster_emb_v,
           cluster_emb_know):
    B, S, _ = x.shape
    tokens = B * S
    x2d = x.reshape(tokens, D_MODEL)

    n_qk = cluster_emb_qk.shape[0] * CLUSTER_SIZE
    n_v = cluster_emb_v.shape[0] * CLUSTER_SIZE
    n_know = cluster_emb_know.shape[0] * CLUSTER_SIZE

    qk_emb_t = neuron_emb[:n_qk].T
    v_emb_t = neuron_emb[n_qk:n_qk + n_v].T
    know_emb_t = neuron_emb[n_qk + n_v:].T

    def mk_wgt(w_h, w_tau_col):
        pad = jnp.zeros((D_MODEL, 128 - D_SPACE - 1), jnp.float32)
        return jnp.concatenate([w_h, w_tau_col, pad], axis=1)

    def mk_bgt(b_h, b_tau_col):
        pad = jnp.zeros((128 - D_SPACE - 1,), jnp.float32)
        return jnp.concatenate([b_h, b_tau_col, pad])[None, :]

    def mk_ex(n_clusters):
        n = n_clusters * CLUSTER_SIZE
        rows = jnp.arange(n_clusters)[:, None]
        cols = jnp.arange(n)[None, :] // CLUSTER_SIZE
        return (rows == cols).astype(jnp.float32)

    gates = []
    auxs = []
    specs = [
        (W_attn[:, 0:D_SPACE], b_attn[0:D_SPACE], W_tau_attn[:, 0:1],
         b_tau_attn[0:1], qk_emb_t, cluster_emb_qk.T, n_qk),
        (W_attn[:, D_SPACE:2 * D_SPACE], b_attn[D_SPACE:2 * D_SPACE],
         W_tau_attn[:, 1:2], b_tau_attn[1:2], qk_emb_t, cluster_emb_qk.T,
         n_qk),
        (W_attn[:, 2 * D_SPACE:], b_attn[2 * D_SPACE:], W_tau_attn[:, 2:3],
         b_tau_attn[2:3], v_emb_t, cluster_emb_v.T, n_v),
        (W_know, b_know, W_tau_know, b_tau_know, know_emb_t,
         cluster_emb_know.T, n_know),
    ]
    for w_h, b_h, w_t, b_t, emb_t, ce_t, n in specs:
        c = n // CLUSTER_SIZE
        out, accc, accn = _run_gate(
            x2d, mk_wgt(w_h, w_t), mk_bgt(b_h, b_t), emb_t, ce_t, mk_ex(c),
            c, n)
        gates.append(out.reshape(B, S, n))
        auxs.append(_aux(accc, tokens, c) + _aux(accn, tokens, n))

    aux = auxs[0] + auxs[1] + auxs[2] + auxs[3]
    return gates[0], gates[1], gates[2], gates[3], aux
